# baseline (speedup 1.0000x reference)
import functools
import jax
import jax.numpy as jnp
from jax import lax
from jax.experimental import pallas as pl
from jax.experimental.pallas import tpu as pltpu
from jax.experimental.pallas import tpu_sc as plsc

HID = 128
ATTN = 32
NLAYER = 3
NENT = 50000
LANES = 16
GW = HID + ATTN
CHUNK = 48


def _prep_body(rela_ref, hqr_ref, pw_ref, pb_ref, wr_ref, wqr_ref, wqrb_ref,
               rel_out, bq_out):
    pw = pw_ref[0]
    pb = pb_ref[0]
    hr = lax.dot_general(rela_ref[...], pw, (((1,), (1,)), ((), ())),
                         preferred_element_type=jnp.float32) + pb
    rel_out[0, :, :HID] = hr
    rel_out[0, :, HID:] = lax.dot_general(
        hr, wr_ref[0], (((1,), (1,)), ((), ())),
        preferred_element_type=jnp.float32) + wqrb_ref[0]
    hqr = lax.dot_general(hqr_ref[...], pw, (((1,), (1,)), ((), ())),
                          preferred_element_type=jnp.float32) + pb
    bq_out[0] = lax.dot_general(hqr, wqr_ref[0], (((1,), (1,)), ((), ())),
                                preferred_element_type=jnp.float32)


def _prep(rela, hqr_raw, proj_W, proj_b, Wr_W, Wqr_W, Wqr_b):
    nrel, llm = rela.shape
    n = hqr_raw.shape[0]
    return pl.pallas_call(
        _prep_body,
        grid=(NLAYER,),
        in_specs=[
            pl.BlockSpec((nrel, llm), lambda i: (0, 0)),
            pl.BlockSpec((n, llm), lambda i: (0, 0)),
            pl.BlockSpec((1, HID, llm), lambda i: (i, 0, 0)),
            pl.BlockSpec((1, 1, HID), lambda i: (i, 0, 0)),
            pl.BlockSpec((1, ATTN, HID), lambda i: (i, 0, 0)),
            pl.BlockSpec((1, ATTN, HID), lambda i: (i, 0, 0)),
            pl.BlockSpec((1, 1, ATTN), lambda i: (i, 0, 0)),
        ],
        out_specs=[
            pl.BlockSpec((1, nrel, GW), lambda i: (i, 0, 0)),
            pl.BlockSpec((1, n, ATTN), lambda i: (i, 0, 0)),
        ],
        out_shape=[
            jax.ShapeDtypeStruct((NLAYER, nrel, GW), jnp.float32),
            jax.ShapeDtypeStruct((NLAYER, n, ATTN), jnp.float32),
        ],
    )(rela, hqr_raw, proj_W, proj_b.reshape(NLAYER, 1, HID), Wr_W, Wqr_W,
      Wqr_b.reshape(NLAYER, 1, ATTN))


def _g0_body(h_ref, ws_ref, g_out):
    h = h_ref[...]
    g_out[:, :HID] = h
    g_out[:, HID:] = lax.dot_general(h, ws_ref[...], (((1,), (1,)), ((), ())),
                                     preferred_element_type=jnp.float32)


def _g0(hidden, Ws):
    n_node = hidden.shape[0]
    blk = 1000
    return pl.pallas_call(
        _g0_body,
        grid=(n_node // blk,),
        in_specs=[
            pl.BlockSpec((blk, HID), lambda i: (i, 0)),
            pl.BlockSpec((ATTN, HID), lambda i: (0, 0)),
        ],
        out_specs=pl.BlockSpec((blk, GW), lambda i: (i, 0)),
        out_shape=jax.ShapeDtypeStruct((n_node, GW), jnp.float32),
    )(hidden, Ws)


def _update_body(acc_ref, h0_ref, wh_ref, wih_ref, whh_ref, bih_ref, bhh_ref,
                 wx_ref, g_out, h0_out):
    agg = acc_ref[0] + acc_ref[1]
    hn = jnp.maximum(
        lax.dot_general(agg, wh_ref[...], (((1,), (1,)), ((), ())),
                        preferred_element_type=jnp.float32), 0.0)
    alive = jnp.sum(hn, axis=-1, keepdims=True)
    mask = jnp.where(alive == 0.0, 0.0, 1.0)
    gi = lax.dot_general(hn, wih_ref[...], (((1,), (1,)), ((), ())),
                         preferred_element_type=jnp.float32) + bih_ref[...]
    h0 = h0_ref[...]
    gh = lax.dot_general(h0, whh_ref[...], (((1,), (1,)), ((), ())),
                         preferred_element_type=jnp.float32) + bhh_ref[...]
    r = jax.nn.sigmoid(gi[:, :HID] + gh[:, :HID])
    z = jax.nn.sigmoid(gi[:, HID:2 * HID] + gh[:, HID:2 * HID])
    cand = jnp.tanh(gi[:, 2 * HID:] + r * gh[:, 2 * HID:])
    h_new = (1.0 - z) * cand + z * h0
    hid = h_new * mask
    g_out[:, :HID] = hid
    g_out[:, HID:] = lax.dot_general(hid, wx_ref[...], (((1,), (1,)), ((), ())),
                                     preferred_element_type=jnp.float32)
    h0_out[...] = hid


def _update(acc2, h0, Wh, Wih, Whh, bih, bhh, Wx):
    n_node = h0.shape[0]
    blk = 1000
    return pl.pallas_call(
        _update_body,
        grid=(n_node // blk,),
        in_specs=[
            pl.BlockSpec((2, blk, HID), lambda i: (0, i, 0)),
            pl.BlockSpec((blk, HID), lambda i: (i, 0)),
            pl.BlockSpec((HID, HID), lambda i: (0, 0)),
            pl.BlockSpec((3 * HID, HID), lambda i: (0, 0)),
            pl.BlockSpec((3 * HID, HID), lambda i: (0, 0)),
            pl.BlockSpec((1, 3 * HID), lambda i: (0, 0)),
            pl.BlockSpec((1, 3 * HID), lambda i: (0, 0)),
            pl.BlockSpec((ATTN, HID), lambda i: (0, 0)),
        ],
        out_specs=[
            pl.BlockSpec((blk, GW), lambda i: (i, 0)),
            pl.BlockSpec((blk, HID), lambda i: (i, 0)),
        ],
        out_shape=[
            jax.ShapeDtypeStruct((n_node, GW), jnp.float32),
            jax.ShapeDtypeStruct((n_node, HID), jnp.float32),
        ],
    )(acc2, h0, Wh, Wih, Whh, bih, bhh, Wx)


def _sc_edge_body(n_node, e_per_w,
                  g_hbm, rel_hbm, bq_hbm, const_hbm, sub_hbm, reli_hbm,
                  obj_hbm, ebi_hbm, out_hbm,
                  sub0, sub1, reli0, reli1, ebi0, ebi1,
                  obj0, obj1, obj2, obj3,
                  gb0, gb1, rb0, rb1, qb0, qb1, mb0, mb1,
                  sub_t, reli_t, obj_t, ebi_t,
                  constv, acc, sem_idx, sem_rows, sem_sc):
    subs, relis, ebis = [sub0, sub1], [reli0, reli1], [ebi0, ebi1]
    objs = [obj0, obj1, obj2, obj3]
    gbufs, rbufs, qbufs, mbufs = [gb0, gb1], [rb0, rb1], [qb0, qb1], [mb0, mb1]
    gbuf, rbuf, bqbuf, msgbuf = gb0, rb0, qb0, mb0
    cid = lax.axis_index("c")
    sid = lax.axis_index("s")
    w = cid * 16 + sid
    base = w * e_per_w
    nfull = e_per_w // CHUNK
    tail = e_per_w % CHUNK
    big = ((n_node // 16 + 7) // 8) * 8
    last = n_node - 15 * big

    pltpu.sync_copy(const_hbm, constv)

    def _zrow(r, _):
        for k in range(HID // LANES):
            msgbuf[r, pl.ds(k * LANES, LANES)] = jnp.zeros((LANES,), jnp.float32)
        return 0
    lax.fori_loop(0, CHUNK, _zrow, 0)

    def _zero_rows(start, size):
        nz = size // CHUNK
        for j in range(nz):
            pltpu.sync_copy(
                msgbuf, acc.at[pl.ds(pl.multiple_of(start + j * CHUNK, 8),
                                     CHUNK)])
        rem = size % CHUNK
        if rem:
            pltpu.sync_copy(
                msgbuf.at[pl.ds(0, rem)],
                acc.at[pl.ds(pl.multiple_of(start + nz * CHUNK, 8), rem)])

    @pl.when(sid < 15)
    def _():
        _zero_rows(pl.multiple_of(sid * big, 8), big)

    @pl.when(sid == 15)
    def _():
        _zero_rows(15 * big, last)

    plsc.subcore_barrier()

    wa0 = constv[pl.ds(0, LANES)]
    wa1 = constv[pl.ds(LANES, LANES)]
    wab = constv[pl.ds(2 * LANES, LANES)]
    lanes = lax.iota(jnp.int32, LANES)
    perms = [(lanes ^ sh) for sh in (8, 4, 2, 1)]
    gdn = lax.GatherDimensionNumbers(offset_dims=(), collapsed_slice_dims=(0,),
                                     start_index_map=(0,))

    def _lane_shuffle(v, p):
        return lax.gather(v, p.reshape(LANES, 1), gdn, (1,),
                          mode=lax.GatherScatterMode.PROMISE_IN_BOUNDS)

    def _make_edge(gb, rb, qb, mb):
        def _edge(e, carry):
            pre0 = jnp.maximum(gb[e, pl.ds(HID, LANES)]
                               + rb[e, pl.ds(HID, LANES)]
                               + qb[e, pl.ds(0, LANES)], 0.0)
            pre1 = jnp.maximum(gb[e, pl.ds(HID + LANES, LANES)]
                               + rb[e, pl.ds(HID + LANES, LANES)]
                               + qb[e, pl.ds(LANES, LANES)], 0.0)
            t = pre0 * wa0 + pre1 * wa1 + wab
            for p in perms:
                t = t + _lane_shuffle(t, p)
            av = 1.0 / (1.0 + jnp.exp(-t))
            for k in range(HID // LANES):
                sl = pl.ds(k * LANES, LANES)
                mb[e, sl] = av * gb[e, sl] * rb[e, sl]
            return carry
        return _edge

    def fire_idx(c, p, s4):
        off = pl.multiple_of(base + c * CHUNK, 8)
        pltpu.async_copy(sub_hbm.at[pl.ds(off, CHUNK)], subs[p], sem_idx)
        pltpu.async_copy(reli_hbm.at[pl.ds(off, CHUNK)], relis[p], sem_idx)
        pltpu.async_copy(ebi_hbm.at[pl.ds(off, CHUNK)], ebis[p], sem_idx)
        pltpu.async_copy(obj_hbm.at[pl.ds(off, CHUNK)], objs[s4], sem_idx)

    def wait_idx(p, s4):
        pltpu.make_async_copy(sub_hbm.at[pl.ds(0, CHUNK)], subs[p],
                              sem_idx).wait()
        pltpu.make_async_copy(reli_hbm.at[pl.ds(0, CHUNK)], relis[p],
                              sem_idx).wait()
        pltpu.make_async_copy(ebi_hbm.at[pl.ds(0, CHUNK)], ebis[p],
                              sem_idx).wait()
        pltpu.make_async_copy(obj_hbm.at[pl.ds(0, CHUNK)], objs[s4],
                              sem_idx).wait()

    def fire_rows(p):
        pltpu.async_copy(g_hbm.at[subs[p]], gbufs[p], sem_rows)
        pltpu.async_copy(rel_hbm.at[relis[p]], rbufs[p], sem_rows)
        pltpu.async_copy(bq_hbm.at[ebis[p]], qbufs[p], sem_rows)

    def wait_rows(p):
        pltpu.make_async_copy(g_hbm.at[subs[p]], gbufs[p], sem_rows).wait()
        pltpu.make_async_copy(rel_hbm.at[relis[p]], rbufs[p], sem_rows).wait()
        pltpu.make_async_copy(bq_hbm.at[ebis[p]], qbufs[p], sem_rows).wait()

    def fire_sc(p, s4):
        pltpu.async_copy(mbufs[p], acc.at[objs[s4]], sem_sc, add=True)

    def wait_sc(p, s4):
        pltpu.make_async_copy(mbufs[p], acc.at[objs[s4]], sem_sc).wait()

    nquad = nfull // 4

    def _step(k, j):
        c = 4 * k + j
        p, q = j % 2, 1 - j % 2

        def _wait_prev_sc():
            wait_sc(p, (j + 2) % 4)
        if j >= 2:
            _wait_prev_sc()
        else:
            pl.when(k >= 1)(_wait_prev_sc)

        def _fire_next_idx():
            fire_idx(c + 1, q, (j + 1) % 4)

        def _wait_fire_next_rows():
            wait_idx(q, (j + 1) % 4)
            fire_rows(q)
        if j < 3:
            _fire_next_idx()
            wait_rows(p)
            _wait_fire_next_rows()
        else:
            pl.when(k < nquad - 1)(_fire_next_idx)
            wait_rows(p)
            pl.when(k < nquad - 1)(_wait_fire_next_rows)
        lax.fori_loop(0, CHUNK,
                      _make_edge(gbufs[p], rbufs[p], qbufs[p], mbufs[p]), 0)
        fire_sc(p, j)

    fire_idx(0, 0, 0)
    wait_idx(0, 0)
    fire_rows(0)

    def _quad(k, _):
        for j in range(4):
            _step(k, j)
        return 0

    lax.fori_loop(0, nquad, _quad, 0)
    wait_sc(0, 2)
    wait_sc(1, 3)

    if tail:
        off = base + nfull * CHUNK
        d1 = pltpu.async_copy(sub_hbm.at[pl.ds(off, tail)], sub_t, sem_idx)
        d2 = pltpu.async_copy(reli_hbm.at[pl.ds(off, tail)], reli_t, sem_idx)
        d3 = pltpu.async_copy(obj_hbm.at[pl.ds(off, tail)], obj_t, sem_idx)
        d4 = pltpu.async_copy(ebi_hbm.at[pl.ds(off, tail)], ebi_t, sem_idx)
        d1.wait(); d2.wait(); d3.wait(); d4.wait()
        g1 = pltpu.async_copy(g_hbm.at[sub_t], gbuf.at[pl.ds(0, tail)],
                              sem_rows)
        g2 = pltpu.async_copy(rel_hbm.at[reli_t], rbuf.at[pl.ds(0, tail)],
                              sem_rows)
        g3 = pltpu.async_copy(bq_hbm.at[ebi_t], bqbuf.at[pl.ds(0, tail)],
                              sem_rows)
        g1.wait(); g2.wait(); g3.wait()
        lax.fori_loop(0, tail, _make_edge(gbuf, rbuf, bqbuf, msgbuf), 0)
        pltpu.sync_copy(msgbuf.at[pl.ds(0, tail)], acc.at[obj_t], add=True)

    plsc.subcore_barrier()

    @pl.when(sid < 15)
    def _():
        start = pl.multiple_of(sid * big, 8)
        pltpu.sync_copy(acc.at[pl.ds(start, big)],
                        out_hbm.at[cid, pl.ds(start, big)])

    @pl.when(sid == 15)
    def _():
        pltpu.sync_copy(acc.at[pl.ds(15 * big, last)],
                        out_hbm.at[cid, pl.ds(15 * big, last)])


def _sc_edges(G, rel_tab, bq, const_v, sub, reli, obj, ebi):
    n_node = G.shape[0]
    n_edge = sub.shape[0]
    e_per_w = n_edge // 32
    tail = e_per_w % CHUNK
    t_sz = max(tail, 8)
    mesh = plsc.VectorSubcoreMesh(core_axis_name="c", subcore_axis_name="s")
    kfn = pl.kernel(
        functools.partial(_sc_edge_body, n_node, e_per_w),
        mesh=mesh,
        compiler_params=pltpu.CompilerParams(use_tc_tiling_on_sc=False),
        out_type=jax.ShapeDtypeStruct((2, n_node, HID), jnp.float32),
        scratch_types=(
            [pltpu.VMEM((CHUNK,), jnp.int32)] * 6
            + [pltpu.VMEM((CHUNK,), jnp.int32)] * 4
            + [pltpu.VMEM((CHUNK, GW), jnp.float32)] * 4
            + [pltpu.VMEM((CHUNK, ATTN), jnp.float32)] * 2
            + [pltpu.VMEM((CHUNK, HID), jnp.float32)] * 2
            + [pltpu.VMEM((t_sz,), jnp.int32)] * 4
            + [pltpu.VMEM((3 * LANES,), jnp.float32)]
            + [pltpu.VMEM_SHARED((n_node, HID), jnp.float32)]
            + [pltpu.SemaphoreType.DMA] * 3
        ),
    )
    return kfn(G, rel_tab, bq, const_v, sub, reli, obj, ebi)


def kernel(q_sub, q_rel, batch_idxs, abs_idxs, query_sub_idxs, edge_batch_idxs,
           edges, rela_embed, proj_W, proj_b, Ws_W, Wr_W, Wqr_W, Wqr_b,
           walpha_W, walpha_b, Wh_W, gru_Wih, gru_Whh, gru_bih, gru_bhh,
           qrel_emb, Wfinal_W):
    n = q_sub.shape[0]
    n_node = batch_idxs.shape[0]

    sub = jnp.asarray(edges[:, 0], jnp.int32)
    reli = jnp.asarray(edges[:, 1], jnp.int32)
    obj = jnp.asarray(edges[:, 2], jnp.int32)
    ebi = jnp.asarray(edge_batch_idxs, jnp.int32)

    hqr_raw = rela_embed[q_rel]
    hidden0 = jnp.zeros((n_node, HID), jnp.float32).at[query_sub_idxs].set(
        qrel_emb[q_rel])
    h0 = jnp.zeros((n_node, HID), jnp.float32)

    rel_tab3, bq3 = _prep(rela_embed, hqr_raw, proj_W, proj_b, Wr_W, Wqr_W,
                          Wqr_b)

    consts = []
    for i in range(NLAYER):
        consts.append(jnp.concatenate([
            walpha_W[i, 0],
            jnp.full((LANES,), walpha_b[i, 0] / LANES, jnp.float32)]))

    wfin_pad = jnp.zeros((ATTN, HID), jnp.float32).at[0].set(Wfinal_W[0])
    bih = gru_bih.reshape(1, 3 * HID)
    bhh = gru_bhh.reshape(1, 3 * HID)

    G = _g0(hidden0, Ws_W[0])
    for i in range(NLAYER):
        acc2 = _sc_edges(G, rel_tab3[i], bq3[i], consts[i], sub, reli, obj,
                         ebi)
        Wx = Ws_W[i + 1] if i + 1 < NLAYER else wfin_pad
        G, h0 = _update(acc2, h0, Wh_W[i], gru_Wih, gru_Whh, bih, bhh, Wx)

    scores = G[:, HID]
    return jnp.zeros((n, NENT), jnp.float32).at[batch_idxs, abs_idxs].set(
        scores)


# speedup vs baseline: 1.0053x; 1.0053x over previous
import functools
import jax
import jax.numpy as jnp
from jax import lax
from jax.experimental import pallas as pl
from jax.experimental.pallas import tpu as pltpu
from jax.experimental.pallas import tpu_sc as plsc

HID = 128
ATTN = 32
NLAYER = 3
NENT = 50000
LANES = 16
GW = HID + ATTN
CHUNK = 48


def _prep_body(rela_ref, hqr_ref, pw_ref, pb_ref, wr_ref, wqr_ref, wqrb_ref,
               rel_out, bq_out):
    pw = pw_ref[0]
    pb = pb_ref[0]
    hr = lax.dot_general(rela_ref[...], pw, (((1,), (1,)), ((), ())),
                         preferred_element_type=jnp.float32) + pb
    rel_out[0, :, :HID] = hr
    rel_out[0, :, HID:] = lax.dot_general(
        hr, wr_ref[0], (((1,), (1,)), ((), ())),
        preferred_element_type=jnp.float32) + wqrb_ref[0]
    hqr = lax.dot_general(hqr_ref[...], pw, (((1,), (1,)), ((), ())),
                          preferred_element_type=jnp.float32) + pb
    bq_out[0] = lax.dot_general(hqr, wqr_ref[0], (((1,), (1,)), ((), ())),
                                preferred_element_type=jnp.float32)


def _prep(rela, hqr_raw, proj_W, proj_b, Wr_W, Wqr_W, Wqr_b):
    nrel, llm = rela.shape
    n = hqr_raw.shape[0]
    return pl.pallas_call(
        _prep_body,
        grid=(NLAYER,),
        in_specs=[
            pl.BlockSpec((nrel, llm), lambda i: (0, 0)),
            pl.BlockSpec((n, llm), lambda i: (0, 0)),
            pl.BlockSpec((1, HID, llm), lambda i: (i, 0, 0)),
            pl.BlockSpec((1, 1, HID), lambda i: (i, 0, 0)),
            pl.BlockSpec((1, ATTN, HID), lambda i: (i, 0, 0)),
            pl.BlockSpec((1, ATTN, HID), lambda i: (i, 0, 0)),
            pl.BlockSpec((1, 1, ATTN), lambda i: (i, 0, 0)),
        ],
        out_specs=[
            pl.BlockSpec((1, nrel, GW), lambda i: (i, 0, 0)),
            pl.BlockSpec((1, n, ATTN), lambda i: (i, 0, 0)),
        ],
        out_shape=[
            jax.ShapeDtypeStruct((NLAYER, nrel, GW), jnp.float32),
            jax.ShapeDtypeStruct((NLAYER, n, ATTN), jnp.float32),
        ],
    )(rela, hqr_raw, proj_W, proj_b.reshape(NLAYER, 1, HID), Wr_W, Wqr_W,
      Wqr_b.reshape(NLAYER, 1, ATTN))


def _g0_body(h_ref, ws_ref, g_out):
    h = h_ref[...]
    g_out[:, :HID] = h
    g_out[:, HID:] = lax.dot_general(h, ws_ref[...], (((1,), (1,)), ((), ())),
                                     preferred_element_type=jnp.float32)


def _g0(hidden, Ws):
    n_node = hidden.shape[0]
    blk = 1000
    return pl.pallas_call(
        _g0_body,
        grid=(n_node // blk,),
        in_specs=[
            pl.BlockSpec((blk, HID), lambda i: (i, 0)),
            pl.BlockSpec((ATTN, HID), lambda i: (0, 0)),
        ],
        out_specs=pl.BlockSpec((blk, GW), lambda i: (i, 0)),
        out_shape=jax.ShapeDtypeStruct((n_node, GW), jnp.float32),
    )(hidden, Ws)


def _update_body(acc_ref, h0_ref, wh_ref, wih_ref, whh_ref, bih_ref, bhh_ref,
                 wx_ref, g_out, h0_out):
    agg = acc_ref[0] + acc_ref[1]
    hn = jnp.maximum(
        lax.dot_general(agg, wh_ref[...], (((1,), (1,)), ((), ())),
                        preferred_element_type=jnp.float32), 0.0)
    alive = jnp.sum(hn, axis=-1, keepdims=True)
    mask = jnp.where(alive == 0.0, 0.0, 1.0)
    gi = lax.dot_general(hn, wih_ref[...], (((1,), (1,)), ((), ())),
                         preferred_element_type=jnp.float32) + bih_ref[...]
    h0 = h0_ref[...]
    gh = lax.dot_general(h0, whh_ref[...], (((1,), (1,)), ((), ())),
                         preferred_element_type=jnp.float32) + bhh_ref[...]
    r = jax.nn.sigmoid(gi[:, :HID] + gh[:, :HID])
    z = jax.nn.sigmoid(gi[:, HID:2 * HID] + gh[:, HID:2 * HID])
    cand = jnp.tanh(gi[:, 2 * HID:] + r * gh[:, 2 * HID:])
    h_new = (1.0 - z) * cand + z * h0
    hid = h_new * mask
    g_out[:, :HID] = hid
    g_out[:, HID:] = lax.dot_general(hid, wx_ref[...], (((1,), (1,)), ((), ())),
                                     preferred_element_type=jnp.float32)
    h0_out[...] = hid


def _update(acc2, h0, Wh, Wih, Whh, bih, bhh, Wx):
    n_node = h0.shape[0]
    blk = 1000
    return pl.pallas_call(
        _update_body,
        grid=(n_node // blk,),
        in_specs=[
            pl.BlockSpec((2, blk, HID), lambda i: (0, i, 0)),
            pl.BlockSpec((blk, HID), lambda i: (i, 0)),
            pl.BlockSpec((HID, HID), lambda i: (0, 0)),
            pl.BlockSpec((3 * HID, HID), lambda i: (0, 0)),
            pl.BlockSpec((3 * HID, HID), lambda i: (0, 0)),
            pl.BlockSpec((1, 3 * HID), lambda i: (0, 0)),
            pl.BlockSpec((1, 3 * HID), lambda i: (0, 0)),
            pl.BlockSpec((ATTN, HID), lambda i: (0, 0)),
        ],
        out_specs=[
            pl.BlockSpec((blk, GW), lambda i: (i, 0)),
            pl.BlockSpec((blk, HID), lambda i: (i, 0)),
        ],
        out_shape=[
            jax.ShapeDtypeStruct((n_node, GW), jnp.float32),
            jax.ShapeDtypeStruct((n_node, HID), jnp.float32),
        ],
    )(acc2, h0, Wh, Wih, Whh, bih, bhh, Wx)


def _sc_edge_body(n_node, e_per_w,
                  g_hbm, rel_hbm, bq_hbm, const_hbm, sub_hbm, reli_hbm,
                  obj_hbm, ebi_hbm, out_hbm,
                  sub0, sub1, reli0, reli1, ebi0, ebi1,
                  obj0, obj1, obj2, obj3,
                  gb0, gb1, rb0, rb1, qb0, qb1, mb0, mb1,
                  sub_t, reli_t, obj_t, ebi_t,
                  constv, acc, sem_idx, sem_rows, sem_sc):
    subs, relis, ebis = [sub0, sub1], [reli0, reli1], [ebi0, ebi1]
    objs = [obj0, obj1, obj2, obj3]
    gbufs, rbufs, qbufs, mbufs = [gb0, gb1], [rb0, rb1], [qb0, qb1], [mb0, mb1]
    gbuf, rbuf, bqbuf, msgbuf = gb0, rb0, qb0, mb0
    cid = lax.axis_index("c")
    sid = lax.axis_index("s")
    w = cid * 16 + sid
    base = w * e_per_w
    nfull = e_per_w // CHUNK
    tail = e_per_w % CHUNK
    big = ((n_node // 16 + 7) // 8) * 8
    last = n_node - 15 * big

    pltpu.sync_copy(const_hbm, constv)

    def _zrow(r, _):
        for k in range(HID // LANES):
            msgbuf[r, pl.ds(k * LANES, LANES)] = jnp.zeros((LANES,), jnp.float32)
        return 0
    lax.fori_loop(0, CHUNK, _zrow, 0)

    def _zero_rows(start, size):
        nz = size // CHUNK
        for j in range(nz):
            pltpu.sync_copy(
                msgbuf, acc.at[pl.ds(pl.multiple_of(start + j * CHUNK, 8),
                                     CHUNK)])
        rem = size % CHUNK
        if rem:
            pltpu.sync_copy(
                msgbuf.at[pl.ds(0, rem)],
                acc.at[pl.ds(pl.multiple_of(start + nz * CHUNK, 8), rem)])

    @pl.when(sid < 15)
    def _():
        _zero_rows(pl.multiple_of(sid * big, 8), big)

    @pl.when(sid == 15)
    def _():
        _zero_rows(15 * big, last)

    plsc.subcore_barrier()

    wa0 = constv[pl.ds(0, LANES)]
    wa1 = constv[pl.ds(LANES, LANES)]
    wab = constv[pl.ds(2 * LANES, LANES)]
    lanes = lax.iota(jnp.int32, LANES)
    perms = [(lanes ^ sh) for sh in (8, 4, 2, 1)]
    gdn = lax.GatherDimensionNumbers(offset_dims=(), collapsed_slice_dims=(0,),
                                     start_index_map=(0,))

    def _lane_shuffle(v, p):
        return lax.gather(v, p.reshape(LANES, 1), gdn, (1,),
                          mode=lax.GatherScatterMode.PROMISE_IN_BOUNDS)

    def _make_edge(gb, rb, qb, mb):
        def _edge(e):
            pre0 = jnp.maximum(gb[e, pl.ds(HID, LANES)]
                               + rb[e, pl.ds(HID, LANES)]
                               + qb[e, pl.ds(0, LANES)], 0.0)
            pre1 = jnp.maximum(gb[e, pl.ds(HID + LANES, LANES)]
                               + rb[e, pl.ds(HID + LANES, LANES)]
                               + qb[e, pl.ds(LANES, LANES)], 0.0)
            t = pre0 * wa0 + pre1 * wa1 + wab
            for p in perms:
                t = t + _lane_shuffle(t, p)
            av = 1.0 / (1.0 + jnp.exp(-t))
            for k in range(HID // LANES):
                sl = pl.ds(k * LANES, LANES)
                mb[e, sl] = av * gb[e, sl] * rb[e, sl]
        return _edge

    def fire_idx(c, p, s4):
        off = pl.multiple_of(base + c * CHUNK, 8)
        pltpu.async_copy(sub_hbm.at[pl.ds(off, CHUNK)], subs[p], sem_idx)
        pltpu.async_copy(reli_hbm.at[pl.ds(off, CHUNK)], relis[p], sem_idx)
        pltpu.async_copy(ebi_hbm.at[pl.ds(off, CHUNK)], ebis[p], sem_idx)
        pltpu.async_copy(obj_hbm.at[pl.ds(off, CHUNK)], objs[s4], sem_idx)

    def wait_idx(p, s4):
        pltpu.make_async_copy(sub_hbm.at[pl.ds(0, CHUNK)], subs[p],
                              sem_idx).wait()
        pltpu.make_async_copy(reli_hbm.at[pl.ds(0, CHUNK)], relis[p],
                              sem_idx).wait()
        pltpu.make_async_copy(ebi_hbm.at[pl.ds(0, CHUNK)], ebis[p],
                              sem_idx).wait()
        pltpu.make_async_copy(obj_hbm.at[pl.ds(0, CHUNK)], objs[s4],
                              sem_idx).wait()

    def fire_rows(p):
        pltpu.async_copy(g_hbm.at[subs[p]], gbufs[p], sem_rows)
        pltpu.async_copy(rel_hbm.at[relis[p]], rbufs[p], sem_rows)
        pltpu.async_copy(bq_hbm.at[ebis[p]], qbufs[p], sem_rows)

    def wait_rows(p):
        pltpu.make_async_copy(g_hbm.at[subs[p]], gbufs[p], sem_rows).wait()
        pltpu.make_async_copy(rel_hbm.at[relis[p]], rbufs[p], sem_rows).wait()
        pltpu.make_async_copy(bq_hbm.at[ebis[p]], qbufs[p], sem_rows).wait()

    def fire_sc(p, s4):
        pltpu.async_copy(mbufs[p], acc.at[objs[s4]], sem_sc, add=True)

    def wait_sc(p, s4):
        pltpu.make_async_copy(mbufs[p], acc.at[objs[s4]], sem_sc).wait()

    nquad = nfull // 4

    def _step(k, j):
        c = 4 * k + j
        p, q = j % 2, 1 - j % 2

        def _wait_prev_sc():
            wait_sc(p, (j + 2) % 4)
        if j >= 2:
            _wait_prev_sc()
        else:
            pl.when(k >= 1)(_wait_prev_sc)

        def _fire_next_idx():
            fire_idx(c + 1, q, (j + 1) % 4)

        def _wait_fire_next_rows():
            wait_idx(q, (j + 1) % 4)
            fire_rows(q)
        if j < 3:
            _fire_next_idx()
            wait_rows(p)
            _wait_fire_next_rows()
        else:
            pl.when(k < nquad - 1)(_fire_next_idx)
            wait_rows(p)
            pl.when(k < nquad - 1)(_wait_fire_next_rows)
        plsc.parallel_loop(0, CHUNK, unroll=4)(
            _make_edge(gbufs[p], rbufs[p], qbufs[p], mbufs[p]))
        fire_sc(p, j)

    fire_idx(0, 0, 0)
    wait_idx(0, 0)
    fire_rows(0)

    def _quad(k, _):
        for j in range(4):
            _step(k, j)
        return 0

    lax.fori_loop(0, nquad, _quad, 0)
    wait_sc(0, 2)
    wait_sc(1, 3)

    if tail:
        off = base + nfull * CHUNK
        d1 = pltpu.async_copy(sub_hbm.at[pl.ds(off, tail)], sub_t, sem_idx)
        d2 = pltpu.async_copy(reli_hbm.at[pl.ds(off, tail)], reli_t, sem_idx)
        d3 = pltpu.async_copy(obj_hbm.at[pl.ds(off, tail)], obj_t, sem_idx)
        d4 = pltpu.async_copy(ebi_hbm.at[pl.ds(off, tail)], ebi_t, sem_idx)
        d1.wait(); d2.wait(); d3.wait(); d4.wait()
        g1 = pltpu.async_copy(g_hbm.at[sub_t], gbuf.at[pl.ds(0, tail)],
                              sem_rows)
        g2 = pltpu.async_copy(rel_hbm.at[reli_t], rbuf.at[pl.ds(0, tail)],
                              sem_rows)
        g3 = pltpu.async_copy(bq_hbm.at[ebi_t], bqbuf.at[pl.ds(0, tail)],
                              sem_rows)
        g1.wait(); g2.wait(); g3.wait()
        plsc.parallel_loop(0, tail, unroll=4)(
            _make_edge(gbuf, rbuf, bqbuf, msgbuf))
        pltpu.sync_copy(msgbuf.at[pl.ds(0, tail)], acc.at[obj_t], add=True)

    plsc.subcore_barrier()

    @pl.when(sid < 15)
    def _():
        start = pl.multiple_of(sid * big, 8)
        pltpu.sync_copy(acc.at[pl.ds(start, big)],
                        out_hbm.at[cid, pl.ds(start, big)])

    @pl.when(sid == 15)
    def _():
        pltpu.sync_copy(acc.at[pl.ds(15 * big, last)],
                        out_hbm.at[cid, pl.ds(15 * big, last)])


def _sc_edges(G, rel_tab, bq, const_v, sub, reli, obj, ebi):
    n_node = G.shape[0]
    n_edge = sub.shape[0]
    e_per_w = n_edge // 32
    tail = e_per_w % CHUNK
    t_sz = max(tail, 8)
    mesh = plsc.VectorSubcoreMesh(core_axis_name="c", subcore_axis_name="s")
    kfn = pl.kernel(
        functools.partial(_sc_edge_body, n_node, e_per_w),
        mesh=mesh,
        compiler_params=pltpu.CompilerParams(use_tc_tiling_on_sc=False),
        out_type=jax.ShapeDtypeStruct((2, n_node, HID), jnp.float32),
        scratch_types=(
            [pltpu.VMEM((CHUNK,), jnp.int32)] * 6
            + [pltpu.VMEM((CHUNK,), jnp.int32)] * 4
            + [pltpu.VMEM((CHUNK, GW), jnp.float32)] * 4
            + [pltpu.VMEM((CHUNK, ATTN), jnp.float32)] * 2
            + [pltpu.VMEM((CHUNK, HID), jnp.float32)] * 2
            + [pltpu.VMEM((t_sz,), jnp.int32)] * 4
            + [pltpu.VMEM((3 * LANES,), jnp.float32)]
            + [pltpu.VMEM_SHARED((n_node, HID), jnp.float32)]
            + [pltpu.SemaphoreType.DMA] * 3
        ),
    )
    return kfn(G, rel_tab, bq, const_v, sub, reli, obj, ebi)


def kernel(q_sub, q_rel, batch_idxs, abs_idxs, query_sub_idxs, edge_batch_idxs,
           edges, rela_embed, proj_W, proj_b, Ws_W, Wr_W, Wqr_W, Wqr_b,
           walpha_W, walpha_b, Wh_W, gru_Wih, gru_Whh, gru_bih, gru_bhh,
           qrel_emb, Wfinal_W):
    n = q_sub.shape[0]
    n_node = batch_idxs.shape[0]

    sub = jnp.asarray(edges[:, 0], jnp.int32)
    reli = jnp.asarray(edges[:, 1], jnp.int32)
    obj = jnp.asarray(edges[:, 2], jnp.int32)
    ebi = jnp.asarray(edge_batch_idxs, jnp.int32)

    hqr_raw = rela_embed[q_rel]
    hidden0 = jnp.zeros((n_node, HID), jnp.float32).at[query_sub_idxs].set(
        qrel_emb[q_rel])
    h0 = jnp.zeros((n_node, HID), jnp.float32)

    rel_tab3, bq3 = _prep(rela_embed, hqr_raw, proj_W, proj_b, Wr_W, Wqr_W,
                          Wqr_b)

    consts = []
    for i in range(NLAYER):
        consts.append(jnp.concatenate([
            walpha_W[i, 0],
            jnp.full((LANES,), walpha_b[i, 0] / LANES, jnp.float32)]))

    wfin_pad = jnp.zeros((ATTN, HID), jnp.float32).at[0].set(Wfinal_W[0])
    bih = gru_bih.reshape(1, 3 * HID)
    bhh = gru_bhh.reshape(1, 3 * HID)

    G = _g0(hidden0, Ws_W[0])
    for i in range(NLAYER):
        acc2 = _sc_edges(G, rel_tab3[i], bq3[i], consts[i], sub, reli, obj,
                         ebi)
        Wx = Ws_W[i + 1] if i + 1 < NLAYER else wfin_pad
        G, h0 = _update(acc2, h0, Wh_W[i], gru_Wih, gru_Whh, bih, bhh, Wx)

    scores = G[:, HID]
    return jnp.zeros((n, NENT), jnp.float32).at[batch_idxs, abs_idxs].set(
        scores)


# speedup vs baseline: 4.2742x; 4.2517x over previous
import functools
import jax
import jax.numpy as jnp
from jax import lax
from jax.experimental import pallas as pl
from jax.experimental.pallas import tpu as pltpu
from jax.experimental.pallas import tpu_sc as plsc

HID = 128
ATTN = 32
NLAYER = 3
NENT = 50000
LANES = 16
GW = HID + ATTN
CHUNK = 48


def _prep_body(rela_ref, hqr_ref, pw_ref, pb_ref, wr_ref, wqr_ref, wqrb_ref,
               rel_out, bq_out):
    pw = pw_ref[0]
    pb = pb_ref[0]
    hr = lax.dot_general(rela_ref[...], pw, (((1,), (1,)), ((), ())),
                         preferred_element_type=jnp.float32) + pb
    rel_out[0, :, :HID] = hr
    rel_out[0, :, HID:] = lax.dot_general(
        hr, wr_ref[0], (((1,), (1,)), ((), ())),
        preferred_element_type=jnp.float32) + wqrb_ref[0]
    hqr = lax.dot_general(hqr_ref[...], pw, (((1,), (1,)), ((), ())),
                          preferred_element_type=jnp.float32) + pb
    bq_out[0] = lax.dot_general(hqr, wqr_ref[0], (((1,), (1,)), ((), ())),
                                preferred_element_type=jnp.float32)


def _prep(rela, hqr_raw, proj_W, proj_b, Wr_W, Wqr_W, Wqr_b):
    nrel, llm = rela.shape
    n = hqr_raw.shape[0]
    return pl.pallas_call(
        _prep_body,
        grid=(NLAYER,),
        in_specs=[
            pl.BlockSpec((nrel, llm), lambda i: (0, 0)),
            pl.BlockSpec((n, llm), lambda i: (0, 0)),
            pl.BlockSpec((1, HID, llm), lambda i: (i, 0, 0)),
            pl.BlockSpec((1, 1, HID), lambda i: (i, 0, 0)),
            pl.BlockSpec((1, ATTN, HID), lambda i: (i, 0, 0)),
            pl.BlockSpec((1, ATTN, HID), lambda i: (i, 0, 0)),
            pl.BlockSpec((1, 1, ATTN), lambda i: (i, 0, 0)),
        ],
        out_specs=[
            pl.BlockSpec((1, nrel, GW), lambda i: (i, 0, 0)),
            pl.BlockSpec((1, n, ATTN), lambda i: (i, 0, 0)),
        ],
        out_shape=[
            jax.ShapeDtypeStruct((NLAYER, nrel, GW), jnp.float32),
            jax.ShapeDtypeStruct((NLAYER, n, ATTN), jnp.float32),
        ],
    )(rela, hqr_raw, proj_W, proj_b.reshape(NLAYER, 1, HID), Wr_W, Wqr_W,
      Wqr_b.reshape(NLAYER, 1, ATTN))


def _g0_body(h_ref, ws_ref, g_out):
    h = h_ref[...]
    g_out[:, :HID] = h
    g_out[:, HID:] = lax.dot_general(h, ws_ref[...], (((1,), (1,)), ((), ())),
                                     preferred_element_type=jnp.float32)


def _g0(hidden, Ws):
    n_node = hidden.shape[0]
    blk = 1000
    return pl.pallas_call(
        _g0_body,
        grid=(n_node // blk,),
        in_specs=[
            pl.BlockSpec((blk, HID), lambda i: (i, 0)),
            pl.BlockSpec((ATTN, HID), lambda i: (0, 0)),
        ],
        out_specs=pl.BlockSpec((blk, GW), lambda i: (i, 0)),
        out_shape=jax.ShapeDtypeStruct((n_node, GW), jnp.float32),
    )(hidden, Ws)


def _update_body(acc_ref, h0_ref, wh_ref, wih_ref, whh_ref, bih_ref, bhh_ref,
                 wx_ref, g_out, h0_out):
    agg = acc_ref[0] + acc_ref[1]
    hn = jnp.maximum(
        lax.dot_general(agg, wh_ref[...], (((1,), (1,)), ((), ())),
                        preferred_element_type=jnp.float32), 0.0)
    alive = jnp.sum(hn, axis=-1, keepdims=True)
    mask = jnp.where(alive == 0.0, 0.0, 1.0)
    gi = lax.dot_general(hn, wih_ref[...], (((1,), (1,)), ((), ())),
                         preferred_element_type=jnp.float32) + bih_ref[...]
    h0 = h0_ref[...]
    gh = lax.dot_general(h0, whh_ref[...], (((1,), (1,)), ((), ())),
                         preferred_element_type=jnp.float32) + bhh_ref[...]
    r = jax.nn.sigmoid(gi[:, :HID] + gh[:, :HID])
    z = jax.nn.sigmoid(gi[:, HID:2 * HID] + gh[:, HID:2 * HID])
    cand = jnp.tanh(gi[:, 2 * HID:] + r * gh[:, 2 * HID:])
    h_new = (1.0 - z) * cand + z * h0
    hid = h_new * mask
    g_out[:, :HID] = hid
    g_out[:, HID:] = lax.dot_general(hid, wx_ref[...], (((1,), (1,)), ((), ())),
                                     preferred_element_type=jnp.float32)
    h0_out[...] = hid


def _update(acc2, h0, Wh, Wih, Whh, bih, bhh, Wx):
    n_node = h0.shape[0]
    blk = 1000
    return pl.pallas_call(
        _update_body,
        grid=(n_node // blk,),
        in_specs=[
            pl.BlockSpec((2, blk, HID), lambda i: (0, i, 0)),
            pl.BlockSpec((blk, HID), lambda i: (i, 0)),
            pl.BlockSpec((HID, HID), lambda i: (0, 0)),
            pl.BlockSpec((3 * HID, HID), lambda i: (0, 0)),
            pl.BlockSpec((3 * HID, HID), lambda i: (0, 0)),
            pl.BlockSpec((1, 3 * HID), lambda i: (0, 0)),
            pl.BlockSpec((1, 3 * HID), lambda i: (0, 0)),
            pl.BlockSpec((ATTN, HID), lambda i: (0, 0)),
        ],
        out_specs=[
            pl.BlockSpec((blk, GW), lambda i: (i, 0)),
            pl.BlockSpec((blk, HID), lambda i: (i, 0)),
        ],
        out_shape=[
            jax.ShapeDtypeStruct((n_node, GW), jnp.float32),
            jax.ShapeDtypeStruct((n_node, HID), jnp.float32),
        ],
    )(acc2, h0, Wh, Wih, Whh, bih, bhh, Wx)


def _bqe_body(ebi_ref, bq_ref, out_ref):
    blk = out_ref.shape[1]
    nb = bq_ref.shape[0]
    oh = (ebi_ref[0, 0][:, None]
          == lax.broadcasted_iota(jnp.int32, (blk, nb), 1)).astype(jnp.float32)
    out_ref[0] = jnp.dot(oh, bq_ref[...], preferred_element_type=jnp.float32)


def _bqe(ebi, bq):
    n_edge = ebi.shape[0]
    blk = 8000
    nblk = n_edge // blk
    nb, attn = bq.shape
    out = pl.pallas_call(
        _bqe_body,
        grid=(nblk,),
        in_specs=[
            pl.BlockSpec((1, 1, blk), lambda i: (i, 0, 0)),
            pl.BlockSpec((nb, attn), lambda i: (0, 0)),
        ],
        out_specs=pl.BlockSpec((1, blk, attn), lambda i: (i, 0, 0)),
        out_shape=jax.ShapeDtypeStruct((nblk, blk, attn), jnp.float32),
    )(ebi.reshape(nblk, 1, blk), bq)
    return out.reshape(n_edge, attn)


def _sc_edge_body(n_node, e_per_w, nq,
                  g_hbm, rel_hbm, bq_hbm, const_hbm, qids_hbm, sub_hbm,
                  reli_hbm, obj_hbm, ebi_hbm, out_hbm,
                  sub0, sub1, reli0, reli1, ebi0, ebi1,
                  obj0, obj1, obj2, obj3,
                  gb0, gb1, rb0, rb1, qb0, qb1, mb0, mb1,
                  sub_t, reli_t, obj_t, ebi_t,
                  constv, qv, acc, sem_idx, sem_rows, sem_sc):
    subs, relis, ebis = [sub0, sub1], [reli0, reli1], [ebi0, ebi1]
    objs = [obj0, obj1, obj2, obj3]
    gbufs, rbufs, mbufs = [gb0, gb1], [rb0, rb1], [mb0, mb1]
    qbufs = [qb0, qb1]
    gbuf, rbuf, bqbuf, msgbuf = gb0, rb0, qb0, mb0
    cid = lax.axis_index("c")
    sid = lax.axis_index("s")
    w = cid * 16 + sid
    base = w * e_per_w
    nfull = e_per_w // CHUNK
    tail = e_per_w % CHUNK
    big = ((n_node // 16 + 7) // 8) * 8
    last = n_node - 15 * big

    pltpu.sync_copy(const_hbm, constv)
    pltpu.sync_copy(qids_hbm, qv)

    def _zrow(r, _):
        for k in range(HID // LANES):
            msgbuf[r, pl.ds(k * LANES, LANES)] = jnp.zeros((LANES,), jnp.float32)
        return 0
    lax.fori_loop(0, CHUNK, _zrow, 0)

    def _zero_rows(start, size):
        nz = size // CHUNK
        for j in range(nz):
            pltpu.sync_copy(
                msgbuf, acc.at[pl.ds(pl.multiple_of(start + j * CHUNK, 8),
                                     CHUNK)])
        rem = size % CHUNK
        if rem:
            pltpu.sync_copy(
                msgbuf.at[pl.ds(0, rem)],
                acc.at[pl.ds(pl.multiple_of(start + nz * CHUNK, 8), rem)])

    @pl.when(sid < 15)
    def _():
        _zero_rows(pl.multiple_of(sid * big, 8), big)

    @pl.when(sid == 15)
    def _():
        _zero_rows(15 * big, last)

    plsc.subcore_barrier()

    wa0 = constv[pl.ds(0, LANES)]
    wa1 = constv[pl.ds(LANES, LANES)]
    wab = constv[pl.ds(2 * LANES, LANES)]
    lanes = lax.iota(jnp.int32, LANES)
    perms = [(lanes ^ sh) for sh in (8, 4, 2, 1)]
    gdn = lax.GatherDimensionNumbers(offset_dims=(), collapsed_slice_dims=(0,),
                                     start_index_map=(0,))

    def _lane_shuffle(v, p):
        return lax.gather(v, p.reshape(LANES, 1), gdn, (1,),
                          mode=lax.GatherScatterMode.PROMISE_IN_BOUNDS)

    qsplats = [_lane_shuffle(qv[...], jnp.full((LANES,), k, jnp.int32))
               for k in range(nq)]

    def _chunk_flag(p):
        hit = None
        for g in range(CHUNK // LANES):
            s16 = subs[p][pl.ds(g * LANES, LANES)]
            m = s16 == qsplats[0]
            for k in range(1, nq):
                m = m | (s16 == qsplats[k])
            hit = m if hit is None else (hit | m)
        t = jnp.where(hit, jnp.int32(1), jnp.int32(0))
        for pp in perms:
            t = t | _lane_shuffle(t, pp)
        return t[0] > 0

    def _make_edge(gb, rb, qb, mb):
        def _edge(e):
            pre0 = jnp.maximum(gb[e, pl.ds(HID, LANES)]
                               + rb[e, pl.ds(HID, LANES)]
                               + qb[e, pl.ds(0, LANES)], 0.0)
            pre1 = jnp.maximum(gb[e, pl.ds(HID + LANES, LANES)]
                               + rb[e, pl.ds(HID + LANES, LANES)]
                               + qb[e, pl.ds(LANES, LANES)], 0.0)
            t = pre0 * wa0 + pre1 * wa1 + wab
            for p in perms:
                t = t + _lane_shuffle(t, p)
            av = 1.0 / (1.0 + jnp.exp(-t))
            for k in range(HID // LANES):
                sl = pl.ds(k * LANES, LANES)
                mb[e, sl] = av * gb[e, sl] * rb[e, sl]
        return _edge

    def fire_idx(c, p, s4):
        off = pl.multiple_of(base + c * CHUNK, 8)
        pltpu.async_copy(sub_hbm.at[pl.ds(off, CHUNK)], subs[p], sem_idx)
        pltpu.async_copy(reli_hbm.at[pl.ds(off, CHUNK)], relis[p], sem_idx)
        pltpu.async_copy(ebi_hbm.at[pl.ds(off, CHUNK)], ebis[p], sem_idx)
        pltpu.async_copy(obj_hbm.at[pl.ds(off, CHUNK)], objs[s4], sem_idx)

    def wait_idx(p, s4):
        pltpu.make_async_copy(sub_hbm.at[pl.ds(0, CHUNK)], subs[p],
                              sem_idx).wait()
        pltpu.make_async_copy(reli_hbm.at[pl.ds(0, CHUNK)], relis[p],
                              sem_idx).wait()
        pltpu.make_async_copy(ebi_hbm.at[pl.ds(0, CHUNK)], ebis[p],
                              sem_idx).wait()
        pltpu.make_async_copy(obj_hbm.at[pl.ds(0, CHUNK)], objs[s4],
                              sem_idx).wait()

    def fire_rows(c, p):
        off = pl.multiple_of(base + c * CHUNK, 8)
        pltpu.async_copy(g_hbm.at[subs[p]], gbufs[p], sem_rows)
        pltpu.async_copy(rel_hbm.at[relis[p]], rbufs[p], sem_rows)
        pltpu.async_copy(bq_hbm.at[pl.ds(off, CHUNK)], qbufs[p], sem_rows)

    def wait_rows(p):
        pltpu.make_async_copy(g_hbm.at[subs[p]], gbufs[p], sem_rows).wait()
        pltpu.make_async_copy(rel_hbm.at[relis[p]], rbufs[p], sem_rows).wait()
        pltpu.make_async_copy(bq_hbm.at[pl.ds(0, CHUNK)], qbufs[p],
                              sem_rows).wait()

    def fire_sc(p, s4):
        pltpu.async_copy(mbufs[p], acc.at[objs[s4]], sem_sc, add=True)

    def wait_sc(p, s4):
        pltpu.make_async_copy(mbufs[p], acc.at[objs[s4]], sem_sc).wait()

    nquad = nfull // 4
    TRUE = jnp.bool_(True)

    def _step(k, j, fring):
        c = 4 * k + j
        p, q = j % 2, 1 - j % 2

        fprev = fring[(j + 2) % 4]
        cond = fprev if j >= 2 else (fprev & (k >= 1))

        def _wait_prev_sc():
            wait_sc(p, (j + 2) % 4)
        pl.when(cond)(_wait_prev_sc)

        def _fire_next_idx():
            fire_idx(c + 1, q, (j + 1) % 4)

        fcur = fring[j]
        next_guard = None if j < 3 else (k < nquad - 1)
        if next_guard is None:
            _fire_next_idx()
        else:
            pl.when(next_guard)(_fire_next_idx)

        pl.when(fcur)(lambda: wait_rows(p))

        if next_guard is None:
            wait_idx(q, (j + 1) % 4)
            fnext = _chunk_flag(q) if nq else TRUE
            pl.when(fnext)(lambda: fire_rows(c + 1, q))
        else:
            pl.when(next_guard)(lambda: wait_idx(q, (j + 1) % 4))
            if nq:
                fnext = _chunk_flag(q) & next_guard
            else:
                fnext = next_guard
            pl.when(fnext)(lambda: fire_rows(c + 1, q))

        def _compute_and_fire():
            plsc.parallel_loop(0, CHUNK, unroll=4)(
                _make_edge(gbufs[p], rbufs[p], qbufs[p], mbufs[p]))
            fire_sc(p, j)
        pl.when(fcur)(_compute_and_fire)

        fring = list(fring)
        fring[(j + 1) % 4] = fnext
        return fring

    fire_idx(0, 0, 0)
    wait_idx(0, 0)
    f0 = _chunk_flag(0) if nq else TRUE
    pl.when(f0)(lambda: fire_rows(0, 0))

    def _quad(k, fring):
        for j in range(4):
            fring = _step(k, j, fring)
        return tuple(fring)

    fring = lax.fori_loop(0, nquad, _quad,
                          (f0, jnp.bool_(False), jnp.bool_(False),
                           jnp.bool_(False)))
    pl.when(fring[2])(lambda: wait_sc(0, 2))
    pl.when(fring[3])(lambda: wait_sc(1, 3))

    if tail:
        off = base + nfull * CHUNK
        d1 = pltpu.async_copy(sub_hbm.at[pl.ds(off, tail)], sub_t, sem_idx)
        d2 = pltpu.async_copy(reli_hbm.at[pl.ds(off, tail)], reli_t, sem_idx)
        d3 = pltpu.async_copy(obj_hbm.at[pl.ds(off, tail)], obj_t, sem_idx)
        d4 = pltpu.async_copy(ebi_hbm.at[pl.ds(off, tail)], ebi_t, sem_idx)
        d1.wait(); d2.wait(); d3.wait(); d4.wait()
        g1 = pltpu.async_copy(g_hbm.at[sub_t], gbuf.at[pl.ds(0, tail)],
                              sem_rows)
        g2 = pltpu.async_copy(rel_hbm.at[reli_t], rbuf.at[pl.ds(0, tail)],
                              sem_rows)
        g3 = pltpu.async_copy(bq_hbm.at[pl.ds(off, tail)],
                              bqbuf.at[pl.ds(0, tail)], sem_rows)
        g1.wait(); g2.wait(); g3.wait()
        plsc.parallel_loop(0, tail, unroll=4)(
            _make_edge(gbuf, rbuf, bqbuf, msgbuf))
        pltpu.sync_copy(msgbuf.at[pl.ds(0, tail)], acc.at[obj_t], add=True)

    plsc.subcore_barrier()

    @pl.when(sid < 15)
    def _():
        start = pl.multiple_of(sid * big, 8)
        pltpu.sync_copy(acc.at[pl.ds(start, big)],
                        out_hbm.at[cid, pl.ds(start, big)])

    @pl.when(sid == 15)
    def _():
        pltpu.sync_copy(acc.at[pl.ds(15 * big, last)],
                        out_hbm.at[cid, pl.ds(15 * big, last)])


def _sc_edges(G, rel_tab, bq, const_v, qids, sub, reli, obj, ebi, nq):
    n_node = G.shape[0]
    n_edge = sub.shape[0]
    e_per_w = n_edge // 32
    tail = e_per_w % CHUNK
    t_sz = max(tail, 8)
    mesh = plsc.VectorSubcoreMesh(core_axis_name="c", subcore_axis_name="s")
    kfn = pl.kernel(
        functools.partial(_sc_edge_body, n_node, e_per_w, nq),
        mesh=mesh,
        compiler_params=pltpu.CompilerParams(use_tc_tiling_on_sc=False),
        out_type=jax.ShapeDtypeStruct((2, n_node, HID), jnp.float32),
        scratch_types=(
            [pltpu.VMEM((CHUNK,), jnp.int32)] * 6
            + [pltpu.VMEM((CHUNK,), jnp.int32)] * 4
            + [pltpu.VMEM((CHUNK, GW), jnp.float32)] * 4
            + [pltpu.VMEM((CHUNK, ATTN), jnp.float32)] * 2
            + [pltpu.VMEM((CHUNK, HID), jnp.float32)] * 2
            + [pltpu.VMEM((t_sz,), jnp.int32)] * 4
            + [pltpu.VMEM((3 * LANES,), jnp.float32)]
            + [pltpu.VMEM((LANES,), jnp.int32)]
            + [pltpu.VMEM_SHARED((n_node, HID), jnp.float32)]
            + [pltpu.SemaphoreType.DMA] * 3
        ),
    )
    return kfn(G, rel_tab, bq, const_v, qids, sub, reli, obj, ebi)


def kernel(q_sub, q_rel, batch_idxs, abs_idxs, query_sub_idxs, edge_batch_idxs,
           edges, rela_embed, proj_W, proj_b, Ws_W, Wr_W, Wqr_W, Wqr_b,
           walpha_W, walpha_b, Wh_W, gru_Wih, gru_Whh, gru_bih, gru_bhh,
           qrel_emb, Wfinal_W):
    n = q_sub.shape[0]
    n_node = batch_idxs.shape[0]

    sub = jnp.asarray(edges[:, 0], jnp.int32)
    reli = jnp.asarray(edges[:, 1], jnp.int32)
    obj = jnp.asarray(edges[:, 2], jnp.int32)
    ebi = jnp.asarray(edge_batch_idxs, jnp.int32)

    hqr_raw = rela_embed[q_rel]
    hidden0 = jnp.zeros((n_node, HID), jnp.float32).at[query_sub_idxs].set(
        qrel_emb[q_rel])
    h0 = jnp.zeros((n_node, HID), jnp.float32)

    rel_tab3, bq3 = _prep(rela_embed, hqr_raw, proj_W, proj_b, Wr_W, Wqr_W,
                          Wqr_b)

    consts = []
    for i in range(NLAYER):
        consts.append(jnp.concatenate([
            walpha_W[i, 0],
            jnp.full((LANES,), walpha_b[i, 0] / LANES, jnp.float32)]))

    qids = jnp.full((LANES,), -1, jnp.int32).at[:n].set(
        query_sub_idxs.astype(jnp.int32))
    wfin_pad = jnp.zeros((ATTN, HID), jnp.float32).at[0].set(Wfinal_W[0])
    bih = gru_bih.reshape(1, 3 * HID)
    bhh = gru_bhh.reshape(1, 3 * HID)

    G = _g0(hidden0, Ws_W[0])
    for i in range(NLAYER):
        acc2 = _sc_edges(G, rel_tab3[i], _bqe(ebi, bq3[i]), consts[i], qids,
                         sub, reli, obj, ebi, nq=n if i == 0 else 0)
        Wx = Ws_W[i + 1] if i + 1 < NLAYER else wfin_pad
        G, h0 = _update(acc2, h0, Wh_W[i], gru_Wih, gru_Whh, bih, bhh, Wx)

    scores = G[:, HID]
    return jnp.zeros((n, NENT), jnp.float32).at[batch_idxs, abs_idxs].set(
        scores)
